# baseline (speedup 1.0000x reference)
import jax
import jax.numpy as jnp
from jax import lax
from jax.experimental import pallas as pl
from jax.experimental.pallas import tpu as pltpu
from jax.experimental.pallas import tpu_sc as plsc

VOCAB = 1000
OUTER = 4096
J = 20
NC, NS, L = 2, 16, 16
JH = J // NC
PLANE_W = VOCAB * OUTER
HALF_W = JH * PLANE_W
TILE_W = HALF_W // NS
IPW = OUTER // NS
ZW = 80000
NZD = TILE_W // ZW
SB = 128
NSB = IPW * JH // SB
GPB = SB // L
TOTAL_W = OUTER * J * VOCAB


def _onehot_body(idx_hbm, zeros_hbm, ones_hbm, out_hbm,
                 idx_v, zbuf, offs, ones_v, semz, sems):
    c = lax.axis_index("c")
    s = lax.axis_index("s")
    pltpu.sync_copy(zeros_hbm, zbuf)
    zbase = c * HALF_W + s * TILE_W

    def zfire(k, u):
        pltpu.make_async_copy(zbuf, out_hbm.at[pl.ds(zbase + k * ZW, ZW)],
                              semz).start()
        return u

    lax.fori_loop(0, NZD, zfire, 0)

    pltpu.sync_copy(idx_hbm.at[pl.ds(s * IPW * J, IPW * J)], idx_v)
    pltpu.sync_copy(ones_hbm, ones_v)
    lane = lax.iota(jnp.int32, L)
    jbase = c * JH

    for p in range(JH):
        for h in range(IPW // SB):
            r = p * (IPW // SB) + h
            for k in range(GPB):
                ir = h * SB + k * L + lane
                v = plsc.load_gather(idx_v, [ir * J + jbase + p])
                i = ir + s * IPW
                off = ((jbase + p) * PLANE_W + (v >> 3) * 32768
                       + (i >> 7) * 1024 + (v & 7) * 128 + (i & 127))
                offs[pl.ds(r * SB + k * L, L)] = off

    def zdrain(k, u):
        pltpu.make_async_copy(zbuf, out_hbm.at[pl.ds(zbase + k * ZW, ZW)],
                              semz).wait()
        return u

    lax.fori_loop(0, NZD, zdrain, 0)
    plsc.subcore_barrier()

    pltpu.make_async_copy(ones_v, out_hbm.at[offs], sems).start()
    pltpu.make_async_copy(ones_v, out_hbm.at[offs], sems).wait()


def kernel(input, idmat):
    del idmat
    idx = input.reshape(-1).astype(jnp.int32)
    zeros_src = jnp.zeros((ZW,), jnp.float32)
    ones_src = jnp.ones((NSB * SB,), jnp.float32)
    mesh = plsc.VectorSubcoreMesh(core_axis_name="c", subcore_axis_name="s",
                                  num_cores=NC)
    f = pl.kernel(
        _onehot_body,
        mesh=mesh,
        compiler_params=pltpu.CompilerParams(needs_layout_passes=False),
        out_type=jax.ShapeDtypeStruct((TOTAL_W,), jnp.float32),
        scratch_types=[
            pltpu.VMEM((IPW * J,), jnp.int32),
            pltpu.VMEM((ZW,), jnp.float32),
            pltpu.VMEM((NSB * SB,), jnp.int32),
            pltpu.VMEM((NSB * SB,), jnp.float32),
            pltpu.SemaphoreType.DMA,
            pltpu.SemaphoreType.DMA,
        ],
    )
    flat = f(idx, zeros_src, ones_src)
    t = flat.reshape(J, VOCAB // 8, OUTER // 128, 8, 128)
    return jnp.transpose(t, (2, 4, 0, 1, 3)).reshape(OUTER, J, VOCAB)


# speedup vs baseline: 1.2209x; 1.2209x over previous
import jax
import jax.numpy as jnp
from jax import lax
from jax.experimental import pallas as pl
from jax.experimental.pallas import tpu as pltpu
from jax.experimental.pallas import tpu_sc as plsc

VOCAB = 1000
OUTER = 4096
J = 20
NC, NS, L = 2, 16, 16
JH = J // NC
PLANE_W = VOCAB * OUTER
HALF_W = JH * PLANE_W
IPW = OUTER // NS
ZW = 64000
NZP = IPW * VOCAB // ZW
SB = 128
BPP = IPW // SB
GPB = SB // L
TOTAL_W = OUTER * J * VOCAB


def _onehot_body(idx_hbm, zeros_hbm, ones_hbm, out_hbm,
                 idx_v, zbuf, offs, ones_v, semz, sems):
    c = lax.axis_index("c")
    s = lax.axis_index("s")
    pltpu.sync_copy(zeros_hbm, zbuf)
    pltpu.sync_copy(idx_hbm.at[pl.ds(s * IPW * J, IPW * J)], idx_v)
    pltpu.sync_copy(ones_hbm, ones_v)

    lane = lax.iota(jnp.int32, L)
    jbase = c * JH
    zbase = c * HALF_W + s * IPW * VOCAB

    def zero_fire(p):
        for k in range(NZP):
            pltpu.make_async_copy(
                zbuf, out_hbm.at[pl.ds(zbase + p * PLANE_W + k * ZW, ZW)],
                semz).start()

    def zero_drain(p):
        for k in range(NZP):
            pltpu.make_async_copy(
                zbuf, out_hbm.at[pl.ds(zbase + p * PLANE_W + k * ZW, ZW)],
                semz).wait()

    def scatter_fire(p):
        for h in range(BPP):
            r = p * BPP + h
            for k in range(GPB):
                ir = h * SB + k * L + lane
                v = plsc.load_gather(idx_v, [ir * J + jbase + p])
                i = ir + s * IPW
                off = ((jbase + p) * PLANE_W + (v >> 3) * 32768
                       + (i >> 7) * 1024 + (v & 7) * 128 + (i & 127))
                offs[r, pl.ds(k * L, L)] = off
            pltpu.make_async_copy(ones_v, out_hbm.at[offs.at[r]], sems).start()

    for p in range(JH):
        zero_fire(p)
        if p > 0:
            scatter_fire(p - 1)
        zero_drain(p)
        plsc.subcore_barrier()
    scatter_fire(JH - 1)

    def sdrain(r, u):
        pltpu.make_async_copy(ones_v, out_hbm.at[offs.at[r]], sems).wait()
        return u

    lax.fori_loop(0, JH * BPP, sdrain, 0)


def kernel(input, idmat):
    del idmat
    idx = input.reshape(-1).astype(jnp.int32)
    zeros_src = jnp.zeros((ZW,), jnp.float32)
    ones_src = jnp.ones((SB,), jnp.float32)
    mesh = plsc.VectorSubcoreMesh(core_axis_name="c", subcore_axis_name="s",
                                  num_cores=NC)
    f = pl.kernel(
        _onehot_body,
        mesh=mesh,
        compiler_params=pltpu.CompilerParams(needs_layout_passes=False),
        out_type=jax.ShapeDtypeStruct((TOTAL_W,), jnp.float32),
        scratch_types=[
            pltpu.VMEM((IPW * J,), jnp.int32),
            pltpu.VMEM((ZW,), jnp.float32),
            pltpu.VMEM((JH * BPP, SB), jnp.int32),
            pltpu.VMEM((SB,), jnp.float32),
            pltpu.SemaphoreType.DMA,
            pltpu.SemaphoreType.DMA,
        ],
    )
    flat = f(idx, zeros_src, ones_src)
    t = flat.reshape(J, VOCAB // 8, OUTER // 128, 8, 128)
    return jnp.transpose(t, (2, 4, 0, 1, 3)).reshape(OUTER, J, VOCAB)
